# baseline (speedup 1.0000x reference)
import functools

import jax
import jax.numpy as jnp
from jax.experimental import pallas as pl
from jax.experimental.pallas import tpu as pltpu

_NB = 2


def _body(words_ref, ps_ref, pt_ref, out_ref, acc_ref, *, nsteps):
    i = pl.program_id(0)

    @pl.when(i == 0)
    def _init():
        acc_ref[0] = 0.0
        acc_ref[1] = 0.0

    row = jax.lax.broadcasted_iota(jnp.int32, (64, 1), 0)
    sh = row & 31
    total = 0.0
    count = 0.0
    for j in range(_NB):
        p = i * _NB + j
        w0 = words_ref[2 * p]
        w1 = words_ref[2 * p + 1]
        w = jnp.where(row < 32, w0, w1)
        bit = (w >> sh) & 1
        m = bit > 0
        safe = jnp.where(m, ps_ref[j], 1.0)
        total += jnp.sum(pt_ref[j] * jnp.log(safe))
        count += jnp.sum(bit.astype(jnp.float32))
    acc_ref[0] += total
    acc_ref[1] += count

    @pl.when(i == nsteps - 1)
    def _fin():
        out_ref[...] = jnp.broadcast_to(-acc_ref[0] / acc_ref[1], (1, 1))


def kernel(ps, pt, bool_masked_pos):
    B, N, K = ps.shape
    pst = jnp.transpose(ps, (1, 0, 2))
    ptt = jnp.transpose(pt, (1, 0, 2))
    m3 = bool_masked_pos.T.astype(jnp.uint32).reshape(N, 2, 32)
    words = (m3 << jnp.arange(32, dtype=jnp.uint32)).sum(
        axis=-1, dtype=jnp.uint32).reshape(-1).astype(jnp.int32)

    nsteps = N // _NB
    grid_spec = pltpu.PrefetchScalarGridSpec(
        num_scalar_prefetch=1,
        grid=(nsteps,),
        in_specs=[
            pl.BlockSpec((_NB, B, K), lambda i, w: (i, 0, 0)),
            pl.BlockSpec((_NB, B, K), lambda i, w: (i, 0, 0)),
        ],
        out_specs=pl.BlockSpec((1, 1), lambda i, w: (0, 0)),
        scratch_shapes=[pltpu.SMEM((2,), jnp.float32)],
    )
    out = pl.pallas_call(
        functools.partial(_body, nsteps=nsteps),
        grid_spec=grid_spec,
        out_shape=jax.ShapeDtypeStruct((1, 1), jnp.float32),
    )(words, pst, ptt)
    return out[0, 0]


# speedup vs baseline: 1.1850x; 1.1850x over previous
import functools

import jax
import jax.numpy as jnp
from jax.experimental import pallas as pl
from jax.experimental.pallas import tpu as pltpu

_NB = 4


def _body(words_ref, ps_ref, pt_ref, out_ref, acc_ref, *, nsteps):
    i = pl.program_id(0)

    @pl.when(i == 0)
    def _init():
        acc_ref[0] = 0.0
        acc_ref[1] = 0.0

    row = jax.lax.broadcasted_iota(jnp.int32, (64, 1), 0)
    sh = row & 31
    total = 0.0
    count = 0.0
    for j in range(_NB):
        p = i * _NB + j
        w0 = words_ref[2 * p]
        w1 = words_ref[2 * p + 1]
        w = jnp.where(row < 32, w0, w1)
        bit = (w >> sh) & 1
        m = bit > 0
        safe = jnp.where(m, ps_ref[j], 1.0)
        total += jnp.sum(pt_ref[j] * jnp.log(safe))
        count += jnp.sum(bit.astype(jnp.float32))
    acc_ref[0] += total
    acc_ref[1] += count

    @pl.when(i == nsteps - 1)
    def _fin():
        out_ref[...] = jnp.broadcast_to(-acc_ref[0] / acc_ref[1], (1, 1))


def kernel(ps, pt, bool_masked_pos):
    B, N, K = ps.shape
    pst = jnp.transpose(ps, (1, 0, 2))
    ptt = jnp.transpose(pt, (1, 0, 2))
    m3 = bool_masked_pos.T.astype(jnp.uint32).reshape(N, 2, 32)
    words = (m3 << jnp.arange(32, dtype=jnp.uint32)).sum(
        axis=-1, dtype=jnp.uint32).reshape(-1).astype(jnp.int32)

    nsteps = N // _NB
    grid_spec = pltpu.PrefetchScalarGridSpec(
        num_scalar_prefetch=1,
        grid=(nsteps,),
        in_specs=[
            pl.BlockSpec((_NB, B, K), lambda i, w: (i, 0, 0)),
            pl.BlockSpec((_NB, B, K), lambda i, w: (i, 0, 0)),
        ],
        out_specs=pl.BlockSpec((1, 1), lambda i, w: (0, 0)),
        scratch_shapes=[pltpu.SMEM((2,), jnp.float32)],
    )
    out = pl.pallas_call(
        functools.partial(_body, nsteps=nsteps),
        grid_spec=grid_spec,
        out_shape=jax.ShapeDtypeStruct((1, 1), jnp.float32),
    )(words, pst, ptt)
    return out[0, 0]
